# baseline (speedup 1.0000x reference)
import functools

import jax
import jax.numpy as jnp
from jax.experimental import pallas as pl
from jax.experimental.pallas import tpu as pltpu

BI = 512
BJ = 1024


def _proj_body(h_ref, W_ref, alr_ref, arr_ref,
               x_ref, el_ref, er_ref, E_ref, Es_ref, F_ref, Fs_ref):
    xv = jnp.dot(h_ref[...], W_ref[...], preferred_element_type=jnp.float32)
    x_ref[...] = xv
    el = jnp.sum(xv * alr_ref[...], axis=1, keepdims=True)
    er = jnp.sum(xv * arr_ref[...], axis=1, keepdims=True)
    el_ref[...] = el
    er_ref[...] = er
    E_ref[...] = jnp.exp(el)
    Es_ref[...] = jnp.exp(0.2 * el)
    F_ref[...] = jnp.exp(er)
    Fs_ref[...] = jnp.exp(0.2 * er)


def _proj(h, W, al, ar):
    N, K = h.shape
    D = W.shape[1]
    alr = al.reshape(1, D)
    arr = ar.reshape(1, D)
    grid = (N // BI,)
    vec = jax.ShapeDtypeStruct((N, 1), jnp.float32)
    out_types = (jax.ShapeDtypeStruct((N, D), jnp.float32),) + (vec,) * 6
    vspec = pl.BlockSpec((BI, 1), lambda i: (i, 0))
    return pl.pallas_call(
        _proj_body,
        grid=grid,
        in_specs=[
            pl.BlockSpec((BI, K), lambda i: (i, 0)),
            pl.BlockSpec((K, D), lambda i: (0, 0)),
            pl.BlockSpec((1, D), lambda i: (0, 0)),
            pl.BlockSpec((1, D), lambda i: (0, 0)),
        ],
        out_specs=(pl.BlockSpec((BI, D), lambda i: (i, 0)),) + (vspec,) * 6,
        out_shape=out_types,
    )(h, W, alr, arr)


def _att_body(nj, fuse_proj, final_softmax,
              adj_ref, el_ref, E_ref, Es_ref, er_ref, F_ref, Fs_ref, xv_ref,
              br_ref, W2_ref, al2r_ref, ar2r_ref,
              *out_and_scratch):
    if fuse_proj:
        (x2_ref, el2_ref, er2_ref, E2_ref, Es2_ref, F2_ref, Fs2_ref,
         acc_ref, dsum_ref) = out_and_scratch
    else:
        out_ref, acc_ref, dsum_ref = out_and_scratch

    j = pl.program_id(1)

    @pl.when(j == 0)
    def _init():
        acc_ref[...] = jnp.zeros_like(acc_ref)
        dsum_ref[...] = jnp.zeros_like(dsum_ref)

    el = el_ref[...]
    er = er_ref[...]
    cond = (el + er) > 0.0
    ai = jnp.where(cond, E_ref[...], Es_ref[...])
    bj = jnp.where(cond, F_ref[...], Fs_ref[...])
    A = ai * bj * adj_ref[...].astype(jnp.float32)
    dsum_ref[...] += jnp.sum(A, axis=1, keepdims=True)
    acc_ref[...] += jnp.dot(A, xv_ref[...], preferred_element_type=jnp.float32)

    @pl.when(j == nj - 1)
    def _finalize():
        denom = jnp.maximum(dsum_ref[...], 1e-12)
        h = acc_ref[...] / denom + br_ref[...]
        if fuse_proj:
            h = jnp.maximum(h, 0.0)
            x2 = jnp.dot(h, W2_ref[...], preferred_element_type=jnp.float32)
            x2_ref[...] = x2
            el2 = jnp.sum(x2 * al2r_ref[...], axis=1, keepdims=True)
            er2 = jnp.sum(x2 * ar2r_ref[...], axis=1, keepdims=True)
            el2_ref[...] = el2
            er2_ref[...] = er2
            E2_ref[...] = jnp.exp(el2)
            Es2_ref[...] = jnp.exp(0.2 * el2)
            F2_ref[...] = jnp.exp(er2)
            Fs2_ref[...] = jnp.exp(0.2 * er2)
        else:
            if final_softmax:
                m = jnp.max(h, axis=1, keepdims=True)
                hs = h - m
                lse = jnp.log(jnp.sum(jnp.exp(hs), axis=1, keepdims=True))
                h = hs - lse
            out_ref[...] = h


def _att(adj_bf, el, E, Es, er_row, F_row, Fs_row, xv, b, W2, al2, ar2,
         fuse_proj, final_softmax):
    N = adj_bf.shape[0]
    D = xv.shape[1]
    ni, nj = N // BI, N // BJ
    br = b.reshape(1, D)
    if fuse_proj:
        D2 = W2.shape[1]
        al2r = al2.reshape(1, D2)
        ar2r = ar2.reshape(1, D2)
    else:
        D2 = D
        W2 = jnp.zeros((1, 1), jnp.float32)
        al2r = jnp.zeros((1, 1), jnp.float32)
        ar2r = jnp.zeros((1, 1), jnp.float32)

    vspec = pl.BlockSpec((BI, 1), lambda i, j: (i, 0))
    in_specs = [
        pl.BlockSpec((BI, BJ), lambda i, j: (i, j)),
        vspec, vspec, vspec,
        pl.BlockSpec((1, BJ), lambda i, j: (0, j)),
        pl.BlockSpec((1, BJ), lambda i, j: (0, j)),
        pl.BlockSpec((1, BJ), lambda i, j: (0, j)),
        pl.BlockSpec((BJ, D), lambda i, j: (j, 0)),
        pl.BlockSpec(br.shape, lambda i, j: (0, 0)),
        pl.BlockSpec(W2.shape, lambda i, j: (0, 0)),
        pl.BlockSpec(al2r.shape, lambda i, j: (0, 0)),
        pl.BlockSpec(ar2r.shape, lambda i, j: (0, 0)),
    ]
    if fuse_proj:
        vec = jax.ShapeDtypeStruct((N, 1), jnp.float32)
        out_shape = (jax.ShapeDtypeStruct((N, D2), jnp.float32),) + (vec,) * 6
        out_specs = ((pl.BlockSpec((BI, D2), lambda i, j: (i, 0)),)
                     + (vspec,) * 6)
    else:
        out_shape = jax.ShapeDtypeStruct((N, D), jnp.float32)
        out_specs = pl.BlockSpec((BI, D), lambda i, j: (i, 0))

    return pl.pallas_call(
        functools.partial(_att_body, nj, fuse_proj, final_softmax),
        grid=(ni, nj),
        in_specs=in_specs,
        out_specs=out_specs,
        out_shape=out_shape,
        scratch_shapes=[
            pltpu.VMEM((BI, D), jnp.float32),
            pltpu.VMEM((BI, 1), jnp.float32),
        ],
        compiler_params=pltpu.CompilerParams(
            dimension_semantics=("arbitrary", "arbitrary")),
    )(adj_bf, el, E, Es, er_row, F_row, Fs_row, xv, br, W2, al2r, ar2r)


def kernel(x, adj, W1, al1, ar1, b1, W2, al2, ar2, b2):
    N = x.shape[0]
    adj_bf = adj.astype(jnp.bfloat16)

    x1, el1, er1, E1, Es1, F1, Fs1 = _proj(x, W1, al1, ar1)
    er1r = er1.reshape(1, N)
    F1r = F1.reshape(1, N)
    Fs1r = Fs1.reshape(1, N)

    (x2, el2, er2, E2, Es2, F2, Fs2) = _att(
        adj_bf, el1, E1, Es1, er1r, F1r, Fs1r, x1, b1, W2, al2, ar2,
        fuse_proj=True, final_softmax=False)

    er2r = er2.reshape(1, N)
    F2r = F2.reshape(1, N)
    Fs2r = Fs2.reshape(1, N)

    out = _att(adj_bf, el2, E2, Es2, er2r, F2r, Fs2r, x2, b2,
               None, None, None, fuse_proj=False, final_softmax=True)
    return out


# speedup vs baseline: 1.6838x; 1.6838x over previous
import functools

import jax
import jax.numpy as jnp
from jax.experimental import pallas as pl
from jax.experimental.pallas import tpu as pltpu

BI = 512


def _scalars(xv, alr, arr):
    el = jnp.sum(xv * alr, axis=1, keepdims=True)
    er = jnp.sum(xv * arr, axis=1, keepdims=True)
    nel = -el
    rho = jnp.exp(0.8 * el).astype(jnp.bfloat16)
    F = jnp.exp(er).astype(jnp.bfloat16)
    Fs = jnp.exp(0.2 * er).astype(jnp.bfloat16)
    return nel, er, rho, F, Fs


def _proj_body(h_ref, W_ref, alr_ref, arr_ref,
               xb_ref, nel_ref, er_ref, rho_ref, F_ref, Fs_ref):
    xv = jnp.dot(h_ref[...], W_ref[...], preferred_element_type=jnp.float32)
    xb_ref[...] = xv.astype(jnp.bfloat16)
    nel, er, rho, F, Fs = _scalars(xv, alr_ref[...], arr_ref[...])
    nel_ref[...] = nel
    er_ref[...] = er
    rho_ref[...] = rho
    F_ref[...] = F
    Fs_ref[...] = Fs


def _proj(h, W, al, ar):
    N, K = h.shape
    D = W.shape[1]
    alr = al.reshape(1, D)
    arr = ar.reshape(1, D)
    fvec = jax.ShapeDtypeStruct((N, 1), jnp.float32)
    bvec = jax.ShapeDtypeStruct((N, 1), jnp.bfloat16)
    vspec = pl.BlockSpec((BI, 1), lambda i: (i, 0))
    return pl.pallas_call(
        _proj_body,
        grid=(N // BI,),
        in_specs=[
            pl.BlockSpec((BI, K), lambda i: (i, 0)),
            pl.BlockSpec((K, D), lambda i: (0, 0)),
            pl.BlockSpec((1, D), lambda i: (0, 0)),
            pl.BlockSpec((1, D), lambda i: (0, 0)),
        ],
        out_specs=(pl.BlockSpec((BI, D), lambda i: (i, 0)),) + (vspec,) * 5,
        out_shape=(jax.ShapeDtypeStruct((N, D), jnp.bfloat16),
                   fvec, fvec, bvec, bvec, bvec),
    )(h, W, alr, arr)


def _att1_body(adj_ref, nel_ref, rho_ref, er_ref, F_ref, Fs_ref, xb_ref,
               br_ref, W2_ref, al2r_ref, ar2r_ref,
               adj8_ref, x2b_ref, nel2_ref, er2_ref, rho2_ref, F2_ref,
               Fs2_ref):
    adj = adj_ref[...]
    adj8_ref[...] = adj.astype(jnp.int8)
    cond = er_ref[...] > nel_ref[...]
    t = rho_ref[...] * F_ref[...]
    B = jnp.where(cond, t, Fs_ref[...]) * adj.astype(jnp.bfloat16)
    dsum = jnp.sum(B.astype(jnp.float32), axis=1, keepdims=True)
    num = jnp.dot(B, xb_ref[...], preferred_element_type=jnp.float32)
    h = num / jnp.maximum(dsum, 1e-12) + br_ref[...]
    h = jnp.maximum(h, 0.0)
    x2 = jnp.dot(h, W2_ref[...], preferred_element_type=jnp.float32)
    x2b_ref[...] = x2.astype(jnp.bfloat16)
    nel2, er2, rho2, F2, Fs2 = _scalars(x2, al2r_ref[...], ar2r_ref[...])
    nel2_ref[...] = nel2
    er2_ref[...] = er2
    rho2_ref[...] = rho2
    F2_ref[...] = F2
    Fs2_ref[...] = Fs2


def _att2_body(adj8_ref, nel_ref, rho_ref, er_ref, F_ref, Fs_ref, xb_ref,
               br_ref, out_ref):
    cond = er_ref[...] > nel_ref[...]
    t = rho_ref[...] * F_ref[...]
    B = jnp.where(cond, t, Fs_ref[...]) * adj8_ref[...].astype(jnp.bfloat16)
    dsum = jnp.sum(B.astype(jnp.float32), axis=1, keepdims=True)
    num = jnp.dot(B, xb_ref[...], preferred_element_type=jnp.float32)
    h = num / jnp.maximum(dsum, 1e-12) + br_ref[...]
    m = jnp.max(h, axis=1, keepdims=True)
    hs = h - m
    lse = jnp.log(jnp.sum(jnp.exp(hs), axis=1, keepdims=True))
    out_ref[...] = hs - lse


def kernel(x, adj, W1, al1, ar1, b1, W2, al2, ar2, b2):
    N = x.shape[0]
    D1 = W1.shape[1]
    D2 = W2.shape[1]

    x1b, nel1, er1, rho1, F1, Fs1 = _proj(x, W1, al1, ar1)
    er1r = er1.reshape(1, N)
    F1r = F1.reshape(1, N)
    Fs1r = Fs1.reshape(1, N)

    fvspec = pl.BlockSpec((BI, 1), lambda i: (i, 0))
    frow = lambda a: pl.BlockSpec((1, N), lambda i: (0, 0))
    fvec = jax.ShapeDtypeStruct((N, 1), jnp.float32)
    bvec = jax.ShapeDtypeStruct((N, 1), jnp.bfloat16)

    adj8, x2b, nel2, er2, rho2, F2, Fs2 = pl.pallas_call(
        _att1_body,
        grid=(N // BI,),
        in_specs=[
            pl.BlockSpec((BI, N), lambda i: (i, 0)),
            fvspec,
            fvspec,
            frow(None), frow(None), frow(None),
            pl.BlockSpec((N, D1), lambda i: (0, 0)),
            pl.BlockSpec((1, D1), lambda i: (0, 0)),
            pl.BlockSpec((D1, D2), lambda i: (0, 0)),
            pl.BlockSpec((1, D2), lambda i: (0, 0)),
            pl.BlockSpec((1, D2), lambda i: (0, 0)),
        ],
        out_specs=(pl.BlockSpec((BI, N), lambda i: (i, 0)),
                   pl.BlockSpec((BI, D2), lambda i: (i, 0)),
                   fvspec, fvspec, fvspec, fvspec, fvspec),
        out_shape=(jax.ShapeDtypeStruct((N, N), jnp.int8),
                   jax.ShapeDtypeStruct((N, D2), jnp.bfloat16),
                   fvec, fvec, bvec, bvec, bvec),
    )(adj, nel1, rho1, er1r, F1r, Fs1r, x1b, b1.reshape(1, D1), W2,
      al2.reshape(1, D2), ar2.reshape(1, D2))

    er2r = er2.reshape(1, N)
    F2r = F2.reshape(1, N)
    Fs2r = Fs2.reshape(1, N)

    out = pl.pallas_call(
        _att2_body,
        grid=(N // BI,),
        in_specs=[
            pl.BlockSpec((BI, N), lambda i: (i, 0)),
            fvspec,
            fvspec,
            frow(None), frow(None), frow(None),
            pl.BlockSpec((N, D2), lambda i: (0, 0)),
            pl.BlockSpec((1, D2), lambda i: (0, 0)),
        ],
        out_specs=pl.BlockSpec((BI, D2), lambda i: (i, 0)),
        out_shape=jax.ShapeDtypeStruct((N, D2), jnp.float32),
    )(adj8, nel2, rho2, er2r, F2r, Fs2r, x2b, b2.reshape(1, D2))
    return out


# speedup vs baseline: 1.7740x; 1.0536x over previous
import jax
import jax.numpy as jnp
from jax.experimental import pallas as pl
from jax.experimental.pallas import tpu as pltpu

BI = 512


def _proj_body(h_ref, W_ref, alr_ref, arr_ref,
               xaug_ref, nel_ref, er_ref, rho_ref, F_ref, Fs_ref):
    xv = jnp.dot(h_ref[...], W_ref[...], preferred_element_type=jnp.float32)
    D = xv.shape[1]
    xaug_ref[:, :D] = xv.astype(jnp.bfloat16)
    lane = jax.lax.broadcasted_iota(jnp.int32, (xv.shape[0], D), 1)
    xaug_ref[:, D:] = jnp.where(lane == 0, 1.0, 0.0).astype(jnp.bfloat16)
    el = jnp.sum(xv * alr_ref[...], axis=1, keepdims=True)
    er = jnp.sum(xv * arr_ref[...], axis=1, keepdims=True)
    nel_ref[...] = (-el).astype(jnp.bfloat16)
    er_ref[...] = er.astype(jnp.bfloat16)
    rho_ref[...] = jnp.exp(0.8 * el).astype(jnp.bfloat16)
    F_ref[...] = jnp.exp(er).astype(jnp.bfloat16)
    Fs_ref[...] = jnp.exp(0.2 * er).astype(jnp.bfloat16)


def _proj(h, W, al, ar):
    N, K = h.shape
    D = W.shape[1]
    bvec = jax.ShapeDtypeStruct((N, 1), jnp.bfloat16)
    vspec = pl.BlockSpec((BI, 1), lambda i: (i, 0))
    return pl.pallas_call(
        _proj_body,
        grid=(N // BI,),
        in_specs=[
            pl.BlockSpec((BI, K), lambda i: (i, 0)),
            pl.BlockSpec((K, D), lambda i: (0, 0)),
            pl.BlockSpec((1, D), lambda i: (0, 0)),
            pl.BlockSpec((1, D), lambda i: (0, 0)),
        ],
        out_specs=(pl.BlockSpec((BI, 2 * D), lambda i: (i, 0)),) + (vspec,) * 5,
        out_shape=(jax.ShapeDtypeStruct((N, 2 * D), jnp.bfloat16),
                   bvec, bvec, bvec, bvec, bvec),
    )(h, W, al.reshape(1, D), ar.reshape(1, D))


def _att1_body(adj_ref, nel_ref, rho_ref, er_ref, F_ref, Fs_ref, xaug_ref,
               br_ref, W2_ref, al2r_ref, ar2r_ref,
               adjb_ref, xaug2_ref, nel2_ref, er2_ref, rho2_ref, F2_ref,
               Fs2_ref):
    adjb = adj_ref[...].astype(jnp.bfloat16)
    adjb_ref[...] = adjb
    cond = er_ref[...] > nel_ref[...]
    t = rho_ref[...] * F_ref[...]
    B = jnp.where(cond, t, Fs_ref[...]) * adjb
    numaug = jnp.dot(B, xaug_ref[...], preferred_element_type=jnp.float32)
    D = numaug.shape[1] // 2
    num = numaug[:, :D]
    denom = numaug[:, D:D + 1]
    recip = 1.0 / jnp.maximum(denom, 1e-12)
    hp = jnp.maximum(num + denom * br_ref[...], 0.0)
    x2p = jnp.dot(hp, W2_ref[...], preferred_element_type=jnp.float32)
    D2 = x2p.shape[1]
    xaug2_ref[:, :D2] = x2p.astype(jnp.bfloat16)
    lane = jax.lax.broadcasted_iota(jnp.int32, (x2p.shape[0], D2), 1)
    xaug2_ref[:, D2:] = jnp.where(lane == 0, denom, 0.0).astype(jnp.bfloat16)
    el2 = jnp.sum(x2p * al2r_ref[...], axis=1, keepdims=True) * recip
    er2 = jnp.sum(x2p * ar2r_ref[...], axis=1, keepdims=True) * recip
    nel2_ref[...] = (-el2).astype(jnp.bfloat16)
    er2_ref[...] = er2.astype(jnp.bfloat16)
    rho2_ref[...] = jnp.exp(0.8 * el2).astype(jnp.bfloat16)
    F2_ref[...] = (jnp.exp(er2) * recip).astype(jnp.bfloat16)
    Fs2_ref[...] = (jnp.exp(0.2 * er2) * recip).astype(jnp.bfloat16)


def _att2_body(adjb_ref, nel_ref, rho_ref, er_ref, F_ref, Fs_ref, xaug_ref,
               br_ref, out_ref):
    cond = er_ref[...] > nel_ref[...]
    t = rho_ref[...] * F_ref[...]
    B = jnp.where(cond, t, Fs_ref[...]) * adjb_ref[...]
    numaug = jnp.dot(B, xaug_ref[...], preferred_element_type=jnp.float32)
    D = numaug.shape[1] // 2
    num = numaug[:, :D]
    d2 = numaug[:, D:D + 1]
    h = num / jnp.maximum(d2, 1e-12) + br_ref[...]
    m = jnp.max(h, axis=1, keepdims=True)
    hs = h - m
    lse = jnp.log(jnp.sum(jnp.exp(hs), axis=1, keepdims=True))
    out_ref[...] = hs - lse


def kernel(x, adj, W1, al1, ar1, b1, W2, al2, ar2, b2):
    N = x.shape[0]
    D1 = W1.shape[1]
    D2 = W2.shape[1]

    xaug1, nel1, er1, rho1, F1, Fs1 = _proj(x, W1, al1, ar1)

    bvspec = pl.BlockSpec((BI, 1), lambda i: (i, 0))
    rowspec = pl.BlockSpec((1, N), lambda i: (0, 0))
    bvec = jax.ShapeDtypeStruct((N, 1), jnp.bfloat16)

    adjb, xaug2, nel2, er2, rho2, F2, Fs2 = pl.pallas_call(
        _att1_body,
        grid=(N // BI,),
        in_specs=[
            pl.BlockSpec((BI, N), lambda i: (i, 0)),
            bvspec,
            bvspec,
            rowspec, rowspec, rowspec,
            pl.BlockSpec((N, 2 * D1), lambda i: (0, 0)),
            pl.BlockSpec((1, D1), lambda i: (0, 0)),
            pl.BlockSpec((D1, D2), lambda i: (0, 0)),
            pl.BlockSpec((1, D2), lambda i: (0, 0)),
            pl.BlockSpec((1, D2), lambda i: (0, 0)),
        ],
        out_specs=(pl.BlockSpec((BI, N), lambda i: (i, 0)),
                   pl.BlockSpec((BI, 2 * D2), lambda i: (i, 0)),
                   bvspec, bvspec, bvspec, bvspec, bvspec),
        out_shape=(jax.ShapeDtypeStruct((N, N), jnp.bfloat16),
                   jax.ShapeDtypeStruct((N, 2 * D2), jnp.bfloat16),
                   bvec, bvec, bvec, bvec, bvec),
    )(adj, nel1, rho1, er1.reshape(1, N), F1.reshape(1, N),
      Fs1.reshape(1, N), xaug1, b1.reshape(1, D1), W2,
      al2.reshape(1, D2), ar2.reshape(1, D2))

    out = pl.pallas_call(
        _att2_body,
        grid=(N // BI,),
        in_specs=[
            pl.BlockSpec((BI, N), lambda i: (i, 0)),
            bvspec,
            bvspec,
            rowspec, rowspec, rowspec,
            pl.BlockSpec((N, 2 * D2), lambda i: (0, 0)),
            pl.BlockSpec((1, D2), lambda i: (0, 0)),
        ],
        out_specs=pl.BlockSpec((BI, D2), lambda i: (i, 0)),
        out_shape=jax.ShapeDtypeStruct((N, D2), jnp.float32),
    )(adjb, nel2, rho2, er2.reshape(1, N), F2.reshape(1, N),
      Fs2.reshape(1, N), xaug2, b2.reshape(1, D2))
    return out


# speedup vs baseline: 1.7741x; 1.0001x over previous
import jax
import jax.numpy as jnp
from jax.experimental import pallas as pl
from jax.experimental.pallas import tpu as pltpu

BI = 512


def _proj_body(h_ref, W_ref, alr_ref, arr_ref,
               xaug_ref, nel_ref, er_ref, rho_ref, F_ref, Fs_ref):
    xv = jnp.dot(h_ref[...], W_ref[...], preferred_element_type=jnp.float32)
    D = xv.shape[1]
    xaug_ref[:, :D] = xv.astype(jnp.bfloat16)
    lane = jax.lax.broadcasted_iota(jnp.int32, (xv.shape[0], D), 1)
    xaug_ref[:, D:] = jnp.where(lane == 0, 1.0, 0.0).astype(jnp.bfloat16)
    el = jnp.sum(xv * alr_ref[...], axis=1, keepdims=True)
    er = jnp.sum(xv * arr_ref[...], axis=1, keepdims=True)
    nel_ref[...] = (-el).astype(jnp.bfloat16)
    er_ref[...] = er.astype(jnp.bfloat16)
    rho_ref[...] = jnp.exp(0.8 * el).astype(jnp.bfloat16)
    F_ref[...] = jnp.exp(er).astype(jnp.bfloat16)
    Fs_ref[...] = jnp.exp(0.2 * er).astype(jnp.bfloat16)


def _proj(h, W, al, ar):
    N, K = h.shape
    D = W.shape[1]
    bvec = jax.ShapeDtypeStruct((N, 1), jnp.bfloat16)
    vspec = pl.BlockSpec((BI, 1), lambda i: (i, 0))
    return pl.pallas_call(
        _proj_body,
        grid=(N // BI,),
        in_specs=[
            pl.BlockSpec((BI, K), lambda i: (i, 0)),
            pl.BlockSpec((K, D), lambda i: (0, 0)),
            pl.BlockSpec((1, D), lambda i: (0, 0)),
            pl.BlockSpec((1, D), lambda i: (0, 0)),
        ],
        out_specs=(pl.BlockSpec((BI, 2 * D), lambda i: (i, 0)),) + (vspec,) * 5,
        out_shape=(jax.ShapeDtypeStruct((N, 2 * D), jnp.bfloat16),
                   bvec, bvec, bvec, bvec, bvec),
        compiler_params=pltpu.CompilerParams(
            dimension_semantics=("parallel",)),
    )(h, W, al.reshape(1, D), ar.reshape(1, D))


def _att1_body(adj_ref, nel_ref, rho_ref, er_ref, F_ref, Fs_ref, xaug_ref,
               br_ref, W2_ref, al2r_ref, ar2r_ref,
               adjb_ref, xaug2_ref, nel2_ref, er2_ref, rho2_ref, F2_ref,
               Fs2_ref):
    adjb = adj_ref[...].astype(jnp.bfloat16)
    adjb_ref[...] = adjb
    cond = er_ref[...] > nel_ref[...]
    t = rho_ref[...] * F_ref[...]
    B = jnp.where(cond, t, Fs_ref[...]) * adjb
    numaug = jnp.dot(B, xaug_ref[...], preferred_element_type=jnp.float32)
    D = numaug.shape[1] // 2
    num = numaug[:, :D]
    denom = numaug[:, D:D + 1]
    recip = 1.0 / jnp.maximum(denom, 1e-12)
    hp = jnp.maximum(num + denom * br_ref[...], 0.0)
    x2p = jnp.dot(hp, W2_ref[...], preferred_element_type=jnp.float32)
    D2 = x2p.shape[1]
    xaug2_ref[:, :D2] = x2p.astype(jnp.bfloat16)
    lane = jax.lax.broadcasted_iota(jnp.int32, (x2p.shape[0], D2), 1)
    xaug2_ref[:, D2:] = jnp.where(lane == 0, denom, 0.0).astype(jnp.bfloat16)
    el2 = jnp.sum(x2p * al2r_ref[...], axis=1, keepdims=True) * recip
    er2 = jnp.sum(x2p * ar2r_ref[...], axis=1, keepdims=True) * recip
    nel2_ref[...] = (-el2).astype(jnp.bfloat16)
    er2_ref[...] = er2.astype(jnp.bfloat16)
    rho2_ref[...] = jnp.exp(0.8 * el2).astype(jnp.bfloat16)
    F2_ref[...] = (jnp.exp(er2) * recip).astype(jnp.bfloat16)
    Fs2_ref[...] = (jnp.exp(0.2 * er2) * recip).astype(jnp.bfloat16)


def _att2_body(adjb_ref, nel_ref, rho_ref, er_ref, F_ref, Fs_ref, xaug_ref,
               br_ref, out_ref):
    cond = er_ref[...] > nel_ref[...]
    t = rho_ref[...] * F_ref[...]
    B = jnp.where(cond, t, Fs_ref[...]) * adjb_ref[...]
    numaug = jnp.dot(B, xaug_ref[...], preferred_element_type=jnp.float32)
    D = numaug.shape[1] // 2
    num = numaug[:, :D]
    d2 = numaug[:, D:D + 1]
    h = num / jnp.maximum(d2, 1e-12) + br_ref[...]
    m = jnp.max(h, axis=1, keepdims=True)
    hs = h - m
    lse = jnp.log(jnp.sum(jnp.exp(hs), axis=1, keepdims=True))
    out_ref[...] = hs - lse


def kernel(x, adj, W1, al1, ar1, b1, W2, al2, ar2, b2):
    N = x.shape[0]
    D1 = W1.shape[1]
    D2 = W2.shape[1]

    xaug1, nel1, er1, rho1, F1, Fs1 = _proj(x, W1, al1, ar1)

    bvspec = pl.BlockSpec((BI, 1), lambda i: (i, 0))
    rowspec = pl.BlockSpec((1, N), lambda i: (0, 0))
    bvec = jax.ShapeDtypeStruct((N, 1), jnp.bfloat16)

    adjb, xaug2, nel2, er2, rho2, F2, Fs2 = pl.pallas_call(
        _att1_body,
        grid=(N // BI,),
        in_specs=[
            pl.BlockSpec((BI, N), lambda i: (i, 0)),
            bvspec,
            bvspec,
            rowspec, rowspec, rowspec,
            pl.BlockSpec((N, 2 * D1), lambda i: (0, 0)),
            pl.BlockSpec((1, D1), lambda i: (0, 0)),
            pl.BlockSpec((D1, D2), lambda i: (0, 0)),
            pl.BlockSpec((1, D2), lambda i: (0, 0)),
            pl.BlockSpec((1, D2), lambda i: (0, 0)),
        ],
        out_specs=(pl.BlockSpec((BI, N), lambda i: (i, 0)),
                   pl.BlockSpec((BI, 2 * D2), lambda i: (i, 0)),
                   bvspec, bvspec, bvspec, bvspec, bvspec),
        out_shape=(jax.ShapeDtypeStruct((N, N), jnp.bfloat16),
                   jax.ShapeDtypeStruct((N, 2 * D2), jnp.bfloat16),
                   bvec, bvec, bvec, bvec, bvec),
        compiler_params=pltpu.CompilerParams(
            dimension_semantics=("parallel",)),
    )(adj, nel1, rho1, er1.reshape(1, N), F1.reshape(1, N),
      Fs1.reshape(1, N), xaug1, b1.reshape(1, D1), W2,
      al2.reshape(1, D2), ar2.reshape(1, D2))

    out = pl.pallas_call(
        _att2_body,
        grid=(N // BI,),
        in_specs=[
            pl.BlockSpec((BI, N), lambda i: (i, 0)),
            bvspec,
            bvspec,
            rowspec, rowspec, rowspec,
            pl.BlockSpec((N, 2 * D2), lambda i: (0, 0)),
            pl.BlockSpec((1, D2), lambda i: (0, 0)),
        ],
        out_specs=pl.BlockSpec((BI, D2), lambda i: (i, 0)),
        out_shape=jax.ShapeDtypeStruct((N, D2), jnp.float32),
        compiler_params=pltpu.CompilerParams(
            dimension_semantics=("parallel",)),
    )(adjb, nel2, rho2, er2.reshape(1, N), F2.reshape(1, N),
      Fs2.reshape(1, N), xaug2, b2.reshape(1, D2))
    return out


# speedup vs baseline: 2.6725x; 1.5064x over previous
import jax
import jax.numpy as jnp
from jax.experimental import pallas as pl
from jax.experimental.pallas import tpu as pltpu

BI = 512
NB = 8


def _scalar_rows(xp, alr, arr, recip):
    el = jnp.sum(xp * alr, axis=1, keepdims=True) * recip
    er = jnp.sum(xp * arr, axis=1, keepdims=True) * recip
    return el, er


def _body(x_ref, adj_ref, W1_ref, al1r_ref, ar1r_ref, b1r_ref,
          W2_ref, al2r_ref, ar2r_ref, b2r_ref, out_ref,
          adj8_s, xaug1_s, nel1_s, rho1_s, er1r_s, F1r_s, Fs1r_s,
          xaug2_s, nel2_s, rho2_s, er2r_s, F2r_s, Fs2r_s):
    s = pl.program_id(0)
    N = adj8_s.shape[1]
    D1 = W1_ref.shape[1]
    D2 = W2_ref.shape[1]

    @pl.when(s < NB)
    def _proj():
        i = s
        rows = pl.ds(i * BI, BI)
        xv = jnp.dot(x_ref[...], W1_ref[...],
                     preferred_element_type=jnp.float32)
        xaug1_s[rows, :D1] = xv.astype(jnp.bfloat16)
        lane = jax.lax.broadcasted_iota(jnp.int32, (BI, D1), 1)
        xaug1_s[rows, D1:] = jnp.where(lane == 0, 1.0, 0.0).astype(
            jnp.bfloat16)
        el, er = _scalar_rows(xv, al1r_ref[...], ar1r_ref[...], 1.0)
        nel1_s[rows, :] = (-el).astype(jnp.bfloat16)
        rho1_s[rows, :] = jnp.exp(0.8 * el).astype(jnp.bfloat16)
        cols = pl.ds(i * BI, BI)
        er1r_s[:, cols] = er.astype(jnp.bfloat16).reshape(1, BI)
        F1r_s[:, cols] = jnp.exp(er).astype(jnp.bfloat16).reshape(1, BI)
        Fs1r_s[:, cols] = jnp.exp(0.2 * er).astype(jnp.bfloat16).reshape(
            1, BI)

    @pl.when(jnp.logical_and(s >= NB, s < 2 * NB))
    def _att1():
        i = s - NB
        rows = pl.ds(i * BI, BI)
        adj = adj_ref[...]
        adj8_s[rows, :] = adj.astype(jnp.int8)
        adjb = adj.astype(jnp.bfloat16)
        cond = er1r_s[...] > nel1_s[rows, :]
        t = rho1_s[rows, :] * F1r_s[...]
        B = jnp.where(cond, t, Fs1r_s[...]) * adjb
        numaug = jnp.dot(B, xaug1_s[...], preferred_element_type=jnp.float32)
        num = numaug[:, :D1]
        denom = numaug[:, D1:D1 + 1]
        recip = 1.0 / jnp.maximum(denom, 1e-12)
        hp = jnp.maximum(num + denom * b1r_ref[...], 0.0)
        x2p = jnp.dot(hp, W2_ref[...], preferred_element_type=jnp.float32)
        xaug2_s[rows, :D2] = x2p.astype(jnp.bfloat16)
        lane = jax.lax.broadcasted_iota(jnp.int32, (BI, D2), 1)
        xaug2_s[rows, D2:] = jnp.where(lane == 0, denom, 0.0).astype(
            jnp.bfloat16)
        el2, er2 = _scalar_rows(x2p, al2r_ref[...], ar2r_ref[...], recip)
        nel2_s[rows, :] = (-el2).astype(jnp.bfloat16)
        rho2_s[rows, :] = jnp.exp(0.8 * el2).astype(jnp.bfloat16)
        cols = pl.ds(i * BI, BI)
        er2r_s[:, cols] = er2.astype(jnp.bfloat16).reshape(1, BI)
        F2r_s[:, cols] = (jnp.exp(er2) * recip).astype(jnp.bfloat16).reshape(
            1, BI)
        Fs2r_s[:, cols] = (jnp.exp(0.2 * er2) * recip).astype(
            jnp.bfloat16).reshape(1, BI)

    @pl.when(s >= 2 * NB)
    def _att2():
        i = s - 2 * NB
        rows = pl.ds(i * BI, BI)
        adjb = adj8_s[rows, :].astype(jnp.bfloat16)
        cond = er2r_s[...] > nel2_s[rows, :]
        t = rho2_s[rows, :] * F2r_s[...]
        B = jnp.where(cond, t, Fs2r_s[...]) * adjb
        numaug = jnp.dot(B, xaug2_s[...], preferred_element_type=jnp.float32)
        num = numaug[:, :D2]
        d2 = numaug[:, D2:D2 + 1]
        h = num / jnp.maximum(d2, 1e-12) + b2r_ref[...]
        m = jnp.max(h, axis=1, keepdims=True)
        hs = h - m
        lse = jnp.log(jnp.sum(jnp.exp(hs), axis=1, keepdims=True))
        out_ref[...] = hs - lse


def kernel(x, adj, W1, al1, ar1, b1, W2, al2, ar2, b2):
    N, K = x.shape
    D1 = W1.shape[1]
    D2 = W2.shape[1]
    bf = jnp.bfloat16

    return pl.pallas_call(
        _body,
        grid=(3 * NB,),
        in_specs=[
            pl.BlockSpec((BI, K), lambda s: (jnp.minimum(s, NB - 1), 0)),
            pl.BlockSpec((BI, N),
                         lambda s: (jnp.clip(s - NB, 0, NB - 1), 0)),
            pl.BlockSpec((K, D1), lambda s: (0, 0)),
            pl.BlockSpec((1, D1), lambda s: (0, 0)),
            pl.BlockSpec((1, D1), lambda s: (0, 0)),
            pl.BlockSpec((1, D1), lambda s: (0, 0)),
            pl.BlockSpec((D1, D2), lambda s: (0, 0)),
            pl.BlockSpec((1, D2), lambda s: (0, 0)),
            pl.BlockSpec((1, D2), lambda s: (0, 0)),
            pl.BlockSpec((1, D2), lambda s: (0, 0)),
        ],
        out_specs=pl.BlockSpec((BI, D2),
                               lambda s: (jnp.clip(s - 2 * NB, 0, NB - 1), 0)),
        out_shape=jax.ShapeDtypeStruct((N, D2), jnp.float32),
        scratch_shapes=[
            pltpu.VMEM((N, N), jnp.int8),
            pltpu.VMEM((N, 2 * D1), bf),
            pltpu.VMEM((N, 1), bf),
            pltpu.VMEM((N, 1), bf),
            pltpu.VMEM((1, N), bf),
            pltpu.VMEM((1, N), bf),
            pltpu.VMEM((1, N), bf),
            pltpu.VMEM((N, 2 * D2), bf),
            pltpu.VMEM((N, 1), bf),
            pltpu.VMEM((N, 1), bf),
            pltpu.VMEM((1, N), bf),
            pltpu.VMEM((1, N), bf),
            pltpu.VMEM((1, N), bf),
        ],
        compiler_params=pltpu.CompilerParams(
            dimension_semantics=("arbitrary",)),
    )(x, adj, W1, al1.reshape(1, D1), ar1.reshape(1, D1), b1.reshape(1, D1),
      W2, al2.reshape(1, D2), ar2.reshape(1, D2), b2.reshape(1, D2))
